# baseline (speedup 1.0000x reference)
import functools

import jax
import jax.numpy as jnp
from jax.experimental import pallas as pl
from jax.experimental.pallas import tpu as pltpu

_K = 8
_CAPACITY_FACTOR = 1.25


def _fused_body(x_ref, w_ref, rw_ref, dm_ref, loss_ref,
                d0_s, kmat_s, hist_s, zsum_s,
                *, n_experts, n_tokens, capacity, bn_a, grid_a):
    i = pl.program_id(0)
    e = n_experts

    @pl.when(i == 0)
    def _init():
        hist_s[...] = jnp.zeros_like(hist_s)
        zsum_s[...] = jnp.zeros_like(zsum_s)

    @pl.when(i < grid_a)
    def _phase_a():
        logits = jax.lax.dot_general(
            w_ref[...], x_ref[...], (((1,), (1,)), ((), ())),
            preferred_element_type=jnp.float32)
        zsum_s[...] += jnp.sum(logits * logits)

        m = jnp.max(logits, axis=0, keepdims=True)
        ex = jnp.exp(logits - m)
        p = ex / jnp.sum(ex, axis=0, keepdims=True)
        rw_ref[...] = jnp.swapaxes(p, 0, 1)

        iota = jax.lax.broadcasted_iota(jnp.int32, p.shape, 0)
        work = p
        kacc = jnp.full(p.shape, float(_K), jnp.float32)
        wsum = jnp.zeros((1, p.shape[1]), jnp.float32)
        hcols = []
        for k in range(_K):
            mk = jnp.max(work, axis=0, keepdims=True)
            idx = jnp.min(jnp.where(work == mk, iota, e), axis=0,
                          keepdims=True)
            oh = (iota == idx).astype(jnp.float32)
            wsum = wsum + mk
            kacc = kacc - (float(_K) - k) * oh
            hcols.append(jnp.sum(oh, axis=1, keepdims=True))
            work = work - oh * (work + 1.0)
        kmat_s[:, pl.ds(i * bn_a, bn_a)] = kacc
        d0_s[:, pl.ds(i * bn_a, bn_a)] = p / (wsum + 1e-8)
        hist_s[...] += jnp.concatenate(hcols, axis=1)

    @pl.when(i == grid_a)
    def _phase_b():
        hist = hist_s[...]
        counts = jnp.zeros((e, 1), jnp.float32)
        kmax = jnp.zeros((e, 1), jnp.float32)
        for k in range(_K):
            a = (counts < capacity).astype(jnp.float32)
            kmax = kmax + a
            counts = counts + hist[:, k:k + 1] * a

        iota_e = jax.lax.broadcasted_iota(jnp.int32, (e, 1), 0)
        minc = jnp.min(counts, axis=0, keepdims=True)
        least = jnp.min(jnp.where(counts == minc, iota_e, e), axis=0,
                        keepdims=True)

        d0 = d0_s[...]
        kmat = kmat_s[...]
        dm = d0 * (kmat < kmax).astype(jnp.float32)
        rs = jnp.sum(dm, axis=0, keepdims=True)
        iota_s = jax.lax.broadcasted_iota(jnp.int32, dm.shape, 0)
        dm = jnp.where((rs == 0.0) & (iota_s == least), 1.0, dm)
        dm = dm / (jnp.sum(dm, axis=0, keepdims=True) + 1e-8)
        dm_ref[...] = jnp.swapaxes(dm, 0, 1)

        counts2 = jnp.sum(dm, axis=1, keepdims=True)
        target = n_tokens * _K / e
        lb = jnp.mean(jnp.square(counts2 / n_tokens - target / n_tokens))
        z = zsum_s[0, 0] / (n_tokens * e)
        loss_ref[...] = jnp.full((1, 1), 0.0, jnp.float32) + (
            0.001 * z + 0.001 * lb)


@jax.jit
def kernel(x, W):
    b, s, d = x.shape
    n = b * s
    e = W.shape[0]
    capacity = int(_CAPACITY_FACTOR * n * _K / e)
    xf = x.reshape(n, d)

    bn_a = 512
    grid_a = n // bn_a
    last = grid_a - 1
    rw, dm, loss = pl.pallas_call(
        functools.partial(_fused_body, n_experts=e, n_tokens=n,
                          capacity=capacity, bn_a=bn_a, grid_a=grid_a),
        grid=(grid_a + 1,),
        in_specs=[
            pl.BlockSpec((bn_a, d), lambda i: (jnp.minimum(i, last), 0)),
            pl.BlockSpec((e, d), lambda i: (0, 0)),
        ],
        out_specs=[
            pl.BlockSpec((bn_a, e), lambda i: (jnp.minimum(i, last), 0)),
            pl.BlockSpec((n, e), lambda i: (0, 0)),
            pl.BlockSpec((1, 1), lambda i: (0, 0)),
        ],
        out_shape=[
            jax.ShapeDtypeStruct((n, e), jnp.float32),
            jax.ShapeDtypeStruct((n, e), jnp.float32),
            jax.ShapeDtypeStruct((1, 1), jnp.float32),
        ],
        scratch_shapes=[
            pltpu.VMEM((e, n), jnp.float32),
            pltpu.VMEM((e, n), jnp.float32),
            pltpu.VMEM((e, _K), jnp.float32),
            pltpu.VMEM((1, 1), jnp.float32),
        ],
    )(xf, W)

    return rw, dm, loss[0, 0]


# speedup vs baseline: 1.0438x; 1.0438x over previous
import functools

import jax
import jax.numpy as jnp
from jax.experimental import pallas as pl
from jax.experimental.pallas import tpu as pltpu

_K = 8
_CAPACITY_FACTOR = 1.25


def _fused_body(x1_ref, x2_ref, w_ref, rw_ref, dm_ref, loss_ref,
                d0_s, kmat_s, hist_s, zsum_s,
                *, n_experts, n_tokens, capacity, bn_a, grid_a):
    i = pl.program_id(0)
    e = n_experts

    @pl.when(i == 0)
    def _init():
        hist_s[...] = jnp.zeros_like(hist_s)
        zsum_s[...] = jnp.zeros_like(zsum_s)

    @pl.when(i < grid_a)
    def _phase_a():
        d2 = x1_ref.shape[1]
        logits = jax.lax.dot_general(
            w_ref[:, :d2], x1_ref[...], (((1,), (1,)), ((), ())),
            preferred_element_type=jnp.float32) + jax.lax.dot_general(
            w_ref[:, d2:], x2_ref[...], (((1,), (1,)), ((), ())),
            preferred_element_type=jnp.float32)
        zsum_s[...] += jnp.sum(logits * logits)

        m = jnp.max(logits, axis=0, keepdims=True)
        ex = jnp.exp(logits - m)
        p = ex / jnp.sum(ex, axis=0, keepdims=True)
        rw_ref[...] = jnp.swapaxes(p, 0, 1)

        iota = jax.lax.broadcasted_iota(jnp.int32, p.shape, 0)
        work = p
        kacc = jnp.full(p.shape, float(_K), jnp.float32)
        wsum = jnp.zeros((1, p.shape[1]), jnp.float32)
        hcols = []
        for k in range(_K):
            mk = jnp.max(work, axis=0, keepdims=True)
            idx = jnp.min(jnp.where(work == mk, iota, e), axis=0,
                          keepdims=True)
            oh = (iota == idx).astype(jnp.float32)
            wsum = wsum + mk
            kacc = kacc - (float(_K) - k) * oh
            hcols.append(jnp.sum(oh, axis=1, keepdims=True))
            work = work - oh * (work + 1.0)
        kmat_s[:, pl.ds(i * bn_a, bn_a)] = kacc
        d0_s[:, pl.ds(i * bn_a, bn_a)] = p / (wsum + 1e-8)
        hist_s[...] += jnp.concatenate(hcols, axis=1)

    @pl.when(i == grid_a)
    def _phase_b():
        hist = hist_s[...]
        counts = jnp.zeros((e, 1), jnp.float32)
        kmax = jnp.zeros((e, 1), jnp.float32)
        for k in range(_K):
            a = (counts < capacity).astype(jnp.float32)
            kmax = kmax + a
            counts = counts + hist[:, k:k + 1] * a

        iota_e = jax.lax.broadcasted_iota(jnp.int32, (e, 1), 0)
        minc = jnp.min(counts, axis=0, keepdims=True)
        least = jnp.min(jnp.where(counts == minc, iota_e, e), axis=0,
                        keepdims=True)

        d0 = d0_s[...]
        kmat = kmat_s[...]
        dm = d0 * (kmat < kmax).astype(jnp.float32)
        rs = jnp.sum(dm, axis=0, keepdims=True)
        iota_s = jax.lax.broadcasted_iota(jnp.int32, dm.shape, 0)
        dm = jnp.where((rs == 0.0) & (iota_s == least), 1.0, dm)
        dm = dm / (jnp.sum(dm, axis=0, keepdims=True) + 1e-8)
        dm_ref[...] = jnp.swapaxes(dm, 0, 1)

        counts2 = jnp.sum(dm, axis=1, keepdims=True)
        target = n_tokens * _K / e
        lb = jnp.mean(jnp.square(counts2 / n_tokens - target / n_tokens))
        z = zsum_s[0, 0] / (n_tokens * e)
        loss_ref[...] = jnp.full((1, 1), 0.0, jnp.float32) + (
            0.001 * z + 0.001 * lb)


@jax.jit
def kernel(x, W):
    b, s, d = x.shape
    n = b * s
    e = W.shape[0]
    capacity = int(_CAPACITY_FACTOR * n * _K / e)
    xf = x.reshape(n, d)

    bn_a = 1024
    grid_a = n // bn_a
    last = grid_a - 1
    rw, dm, loss = pl.pallas_call(
        functools.partial(_fused_body, n_experts=e, n_tokens=n,
                          capacity=capacity, bn_a=bn_a, grid_a=grid_a),
        grid=(grid_a + 1,),
        in_specs=[
            pl.BlockSpec((bn_a, d // 2), lambda i: (jnp.minimum(i, last), 0)),
            pl.BlockSpec((bn_a, d // 2), lambda i: (jnp.minimum(i, last), 1)),
            pl.BlockSpec((e, d), lambda i: (0, 0)),
        ],
        out_specs=[
            pl.BlockSpec((bn_a, e), lambda i: (jnp.minimum(i, last), 0)),
            pl.BlockSpec((n, e), lambda i: (0, 0)),
            pl.BlockSpec((1, 1), lambda i: (0, 0)),
        ],
        out_shape=[
            jax.ShapeDtypeStruct((n, e), jnp.float32),
            jax.ShapeDtypeStruct((n, e), jnp.float32),
            jax.ShapeDtypeStruct((1, 1), jnp.float32),
        ],
        scratch_shapes=[
            pltpu.VMEM((e, n), jnp.float32),
            pltpu.VMEM((e, n), jnp.float32),
            pltpu.VMEM((e, _K), jnp.float32),
            pltpu.VMEM((1, 1), jnp.float32),
        ],
    )(xf, xf, W)

    return rw, dm, loss[0, 0]


# speedup vs baseline: 1.0562x; 1.0119x over previous
import functools

import jax
import jax.numpy as jnp
from jax.experimental import pallas as pl
from jax.experimental.pallas import tpu as pltpu

_K = 8
_CAPACITY_FACTOR = 1.25


def _fused_body(x_ref, w_ref, rw_ref, dm_ref, loss_ref,
                d0_s, kmat_s, hist_s, zsum_s,
                *, n_experts, n_tokens, capacity, bn_a, grid_a):
    i = pl.program_id(0)
    e = n_experts

    @pl.when(i == 0)
    def _init():
        hist_s[...] = jnp.zeros_like(hist_s)
        zsum_s[...] = jnp.zeros_like(zsum_s)

    @pl.when(i < grid_a)
    def _phase_a():
        logits = jax.lax.dot_general(
            w_ref[...], x_ref[...], (((1,), (1,)), ((), ())),
            preferred_element_type=jnp.float32)
        zsum_s[...] += jnp.sum(logits * logits)

        m = jnp.max(logits, axis=0, keepdims=True)
        ex = jnp.exp(logits - m)
        p = ex / jnp.sum(ex, axis=0, keepdims=True)
        rw_ref[...] = jnp.swapaxes(p, 0, 1)

        iota = jax.lax.broadcasted_iota(jnp.int32, p.shape, 0)
        work = p
        kacc = jnp.full(p.shape, float(_K), jnp.float32)
        wsum = jnp.zeros((1, p.shape[1]), jnp.float32)
        hcols = []
        for k in range(_K):
            mk = jnp.max(work, axis=0, keepdims=True)
            idx = jnp.min(jnp.where(work == mk, iota, e), axis=0,
                          keepdims=True)
            oh = (iota == idx).astype(jnp.float32)
            wsum = wsum + mk
            kacc = kacc - (float(_K) - k) * oh
            hcols.append(jnp.sum(oh, axis=1, keepdims=True))
            work = work - oh * (work + 1.0)
        kmat_s[:, pl.ds(i * bn_a, bn_a)] = kacc
        d0_s[:, pl.ds(i * bn_a, bn_a)] = p / (wsum + 1e-8)
        hist_s[...] += jnp.concatenate(hcols, axis=1)

    @pl.when(i == grid_a)
    def _phase_b():
        hist = hist_s[...]
        counts = jnp.zeros((e, 1), jnp.float32)
        kmax = jnp.zeros((e, 1), jnp.float32)
        for k in range(_K):
            a = (counts < capacity).astype(jnp.float32)
            kmax = kmax + a
            counts = counts + hist[:, k:k + 1] * a

        iota_e = jax.lax.broadcasted_iota(jnp.int32, (e, 1), 0)
        minc = jnp.min(counts, axis=0, keepdims=True)
        least = jnp.min(jnp.where(counts == minc, iota_e, e), axis=0,
                        keepdims=True)

        d0 = d0_s[...]
        kmat = kmat_s[...]
        dm = d0 * (kmat < kmax).astype(jnp.float32)
        rs = jnp.sum(dm, axis=0, keepdims=True)
        iota_s = jax.lax.broadcasted_iota(jnp.int32, dm.shape, 0)
        dm = jnp.where((rs == 0.0) & (iota_s == least), 1.0, dm)
        dm = dm / (jnp.sum(dm, axis=0, keepdims=True) + 1e-8)
        dm_ref[...] = jnp.swapaxes(dm, 0, 1)

        counts2 = jnp.sum(dm, axis=1, keepdims=True)
        target = n_tokens * _K / e
        lb = jnp.mean(jnp.square(counts2 / n_tokens - target / n_tokens))
        z = zsum_s[0, 0] / (n_tokens * e)
        loss_ref[...] = jnp.full((1, 1), 0.0, jnp.float32) + (
            0.001 * z + 0.001 * lb)


@jax.jit
def kernel(x, W):
    b, s, d = x.shape
    n = b * s
    e = W.shape[0]
    capacity = int(_CAPACITY_FACTOR * n * _K / e)
    xf = x.reshape(n, d)

    bn_a = 1024
    grid_a = n // bn_a
    last = grid_a - 1
    rw, dm, loss = pl.pallas_call(
        functools.partial(_fused_body, n_experts=e, n_tokens=n,
                          capacity=capacity, bn_a=bn_a, grid_a=grid_a),
        grid=(grid_a + 1,),
        in_specs=[
            pl.BlockSpec((bn_a, d), lambda i: (jnp.minimum(i, last), 0)),
            pl.BlockSpec((e, d), lambda i: (0, 0)),
        ],
        out_specs=[
            pl.BlockSpec((bn_a, e), lambda i: (jnp.minimum(i, last), 0)),
            pl.BlockSpec((n, e), lambda i: (0, 0)),
            pl.BlockSpec((1, 1), lambda i: (0, 0)),
        ],
        out_shape=[
            jax.ShapeDtypeStruct((n, e), jnp.float32),
            jax.ShapeDtypeStruct((n, e), jnp.float32),
            jax.ShapeDtypeStruct((1, 1), jnp.float32),
        ],
        scratch_shapes=[
            pltpu.VMEM((e, n), jnp.float32),
            pltpu.VMEM((e, n), jnp.float32),
            pltpu.VMEM((e, _K), jnp.float32),
            pltpu.VMEM((1, 1), jnp.float32),
        ],
    )(xf, W)

    return rw, dm, loss[0, 0]
